# Initial kernel scaffold; baseline (speedup 1.0000x reference)
import jax
import jax.numpy as jnp
from jax.experimental import pallas as pl


def kernel(xyz, features, params):
    raise NotImplementedError("write your pallas kernel here")


# speedup vs baseline: 6.9799x; 6.9799x over previous
import functools
import math

import jax
import jax.numpy as jnp
from jax import lax
from jax.experimental import pallas as pl
from jax.experimental.pallas import tpu as pltpu
from jax.experimental.pallas import tpu_sc as plsc

B, N = 8, 2048
DP, DM, K, H = 64, 128, 16, 4
HD = DM // H

QA = 256
QB = 128
RB = QB * K


def _stage_a_body(xyzq_ref, xyzt_ref, feat_ref, fc1_ref, fc1b_ref,
                  wq_ref, wk_ref, wv_ref, q_ref, tbl_ref, knn_ref, rel_ref):
    b = pl.program_id(0)
    x = jnp.dot(feat_ref[0], fc1_ref[...]) + fc1b_ref[...]
    q_ref[0] = jnp.dot(x, wq_ref[...])
    xk = jnp.dot(x, wk_ref[...])
    xv = jnp.dot(x, wv_ref[...])
    tbl_ref[0] = jnp.concatenate([xk, xv], axis=1)

    xq = xyzq_ref[0]
    xt = xyzt_ref[0]
    qn = jnp.sum(xq * xq, axis=1, keepdims=True)
    kn = jnp.sum(xt * xt, axis=0, keepdims=True)
    cross = lax.dot_general(xq, xt, (((1,), (0,)), ((), ())),
                            precision=lax.Precision.DEFAULT)
    d = qn + kn - 2.0 * cross

    iota = lax.broadcasted_iota(jnp.int32, (QA, N), 1)
    iok = lax.broadcasted_iota(jnp.int32, (QA, K), 1)
    iok3 = lax.broadcasted_iota(jnp.int32, (QA, K, 3), 1)
    knn = jnp.zeros((QA, K), jnp.int32)
    rel = jnp.zeros((QA, K, 3), jnp.float32)
    for j in range(K):
        m = jnp.min(d, axis=1, keepdims=True)
        idx = jnp.min(jnp.where(d == m, iota, N), axis=1, keepdims=True)
        knn = jnp.where(iok == j, idx, knn)
        hit = iota == idx
        nbx = lax.dot_general(jnp.where(hit, 1.0, 0.0), xt,
                              (((1,), (1,)), ((), ())),
                              precision=lax.Precision.HIGHEST)
        rel = jnp.where(iok3 == j, (xq - nbx)[:, None, :], rel)
        d = jnp.where(hit, jnp.float32(jnp.inf), d)
    knn_ref[0] = knn + b * N
    rel_ref[0] = rel


def _stage_a(xyz, xyzt, feats, p):
    grid = (B, N // QA)
    out = pl.pallas_call(
        _stage_a_body,
        grid=grid,
        in_specs=[
            pl.BlockSpec((1, QA, 3), lambda b, i: (b, i, 0)),
            pl.BlockSpec((1, 3, N), lambda b, i: (b, 0, 0)),
            pl.BlockSpec((1, QA, DP), lambda b, i: (b, i, 0)),
            pl.BlockSpec((DP, DM), lambda b, i: (0, 0)),
            pl.BlockSpec((1, DM), lambda b, i: (0, 0)),
            pl.BlockSpec((DM, DM), lambda b, i: (0, 0)),
            pl.BlockSpec((DM, DM), lambda b, i: (0, 0)),
            pl.BlockSpec((DM, DM), lambda b, i: (0, 0)),
        ],
        out_specs=[
            pl.BlockSpec((1, QA, DM), lambda b, i: (b, i, 0)),
            pl.BlockSpec((1, QA, 2 * DM), lambda b, i: (b, i, 0)),
            pl.BlockSpec((1, QA, K), lambda b, i: (b, i, 0)),
            pl.BlockSpec((1, QA, K, 3), lambda b, i: (b, i, 0, 0)),
        ],
        out_shape=[
            jax.ShapeDtypeStruct((B, N, DM), jnp.float32),
            jax.ShapeDtypeStruct((B, N, 2 * DM), jnp.float32),
            jax.ShapeDtypeStruct((B, N, K), jnp.int32),
            jax.ShapeDtypeStruct((B, N, K, 3), jnp.float32),
        ],
    )(xyz, xyzt, feats, p['fc1_w'], p['fc1_b'].reshape(1, DM),
      p['wq'], p['wk'], p['wv'])
    return out


def _sc_gather(tbl2, idx3, nc, nw):
    tot = B * N * K
    per_w = tot // nw
    ch = 128
    nch = per_w // ch

    mesh = plsc.VectorSubcoreMesh(core_axis_name="c", subcore_axis_name="s")

    @functools.partial(
        pl.kernel,
        mesh=mesh,
        out_type=jax.ShapeDtypeStruct((tot, 2 * DM), jnp.float32),
        scratch_types=[
            pltpu.VMEM((nch, ch), jnp.int32),
            pltpu.VMEM((ch, 2 * DM), jnp.float32),
            pltpu.VMEM((ch, 2 * DM), jnp.float32),
            pltpu.SemaphoreType.DMA,
            pltpu.SemaphoreType.DMA,
        ],
    )
    def k(tbl_hbm, idx_hbm, gkv_hbm, idx_v, buf0, buf1, s0, s1):
        wid = lax.axis_index("s") * nc + lax.axis_index("c")
        pltpu.sync_copy(idx_hbm.at[wid], idx_v)
        bufs = (buf0, buf1)
        sems = (s0, s1)
        pltpu.async_copy(tbl_hbm.at[idx_v.at[0]], buf0, s0)

        def body(i, carry):
            for t in range(2):
                c = 2 * i + t
                buf, sem = bufs[t], sems[t]
                nbuf, nsem = bufs[1 - t], sems[1 - t]
                nc_ = c + 1

                @pl.when(nc_ < nch)
                def _():
                    pltpu.async_copy(tbl_hbm.at[idx_v.at[nc_]], nbuf, nsem)

                pltpu.make_async_copy(tbl_hbm.at[idx_v.at[c]], buf, sem).wait()
                row0 = wid * per_w + c * ch
                pltpu.sync_copy(buf, gkv_hbm.at[pl.ds(row0, ch)])
            return carry

        lax.fori_loop(0, nch // 2, body, 0)

    return k(tbl2, idx3)


def _layernorm(x, g, b):
    m = jnp.mean(x, axis=1, keepdims=True)
    c = x - m
    v = jnp.mean(c * c, axis=1, keepdims=True)
    return c * lax.rsqrt(v + 1e-5) * g + b


def _stage_b_body(q_ref, gkv_ref, rel_ref, feat_ref,
                  fd1_ref, fd1b_ref, fd2_ref, fd2b_ref,
                  fg1_ref, fg1b_ref, fg2_ref, fg2b_ref,
                  proj_ref, n1g_ref, n1b_ref,
                  fc2_ref, fc2b_ref, n2g_ref, n2b_ref,
                  attn_ref, res_ref):
    kk = gkv_ref[:, :DM]
    vv = gkv_ref[:, DM:]

    rel = rel_ref[0].reshape(RB, 3)
    p1 = jax.nn.relu(
        lax.dot_general(rel, fd1_ref[...], (((1,), (0,)), ((), ())))
        + fd1b_ref[...])
    pos = jnp.dot(p1, fd2_ref[...]) + fd2b_ref[...]

    qq = jnp.broadcast_to(q_ref[0][:, None, :], (QB, K, DM)).reshape(RB, DM)
    h = qq - kk + pos
    a = jax.nn.relu(jnp.dot(h, fg1_ref[...]) + fg1b_ref[...])
    a = jnp.dot(a, fg2_ref[...]) + fg2b_ref[...]
    a = a * jnp.float32(1.0 / math.sqrt(HD))
    a3 = a.reshape(QB, K, DM)
    m = jnp.max(a3, axis=1, keepdims=True)
    e = jnp.exp(a3 - m)
    s = jnp.sum(e, axis=1, keepdims=True)
    attn = e / s

    for hh in range(H):
        attn_ref[0, hh] = attn[:, :, hh * HD:(hh + 1) * HD]

    w = attn * (vv + pos).reshape(QB, K, DM)
    resv = jnp.sum(w, axis=1)
    r1 = _layernorm(jnp.dot(resv, proj_ref[...]), n1g_ref[...], n1b_ref[...])
    r2 = jnp.dot(r1, fc2_ref[...]) + fc2b_ref[...]
    r2 = _layernorm(r2, n2g_ref[...], n2b_ref[...]) + feat_ref[0]
    res_ref[0] = r2


def _stage_b(q, gkv, rel, feats, wp):
    grid = (B, N // QB)
    nb = N // QB
    full = lambda r, c: pl.BlockSpec((r, c), lambda b, i: (0, 0))
    out = pl.pallas_call(
        _stage_b_body,
        grid=grid,
        in_specs=[
            pl.BlockSpec((1, QB, DM), lambda b, i: (b, i, 0)),
            pl.BlockSpec((RB, 2 * DM), lambda b, i: (b * nb + i, 0)),
            pl.BlockSpec((1, QB, K, 3), lambda b, i: (b, i, 0, 0)),
            pl.BlockSpec((1, QB, DP), lambda b, i: (b, i, 0)),
            full(3, DM), full(1, DM), full(DM, DM), full(1, DM),
            full(DM, DM), full(1, DM), full(DM, DM), full(1, DM),
            full(DM, DM), full(1, DM), full(1, DM),
            full(DM, DP), full(1, DP), full(1, DP), full(1, DP),
        ],
        out_specs=[
            pl.BlockSpec((1, H, QB, K, HD), lambda b, i: (b, 0, i, 0, 0)),
            pl.BlockSpec((1, QB, DP), lambda b, i: (b, i, 0)),
        ],
        out_shape=[
            jax.ShapeDtypeStruct((B, H, N, K, HD), jnp.float32),
            jax.ShapeDtypeStruct((B, N, DP), jnp.float32),
        ],
    )(q, gkv, rel, feats, *wp)
    return out


def kernel(xyz, features, params):
    p = params
    xyzt = xyz.transpose(0, 2, 1)
    q, tbl, knn, rel = _stage_a(xyz, xyzt, features, p)

    info = plsc.get_sparse_core_info()
    nc, ns = info.num_cores, info.num_subcores
    nw = nc * ns
    tbl2 = tbl.reshape(B * N, 2 * DM)
    idx3 = knn.reshape(nw, (B * N * K) // (nw * 128), 128)
    gkv = _sc_gather(tbl2, idx3, nc, nw)

    z = jnp.zeros((HD, HD), jnp.float32)
    def bd(w):
        rows = []
        for i in range(H):
            rows.append(jnp.concatenate(
                [w if i == j else z for j in range(H)], axis=1))
        return jnp.concatenate(rows, axis=0)
    wp = (
        p['fd1_w'], p['fd1_b'].reshape(1, DM), p['fd2_w'],
        p['fd2_b'].reshape(1, DM),
        bd(p['fg1_w']), jnp.tile(p['fg1_b'], H).reshape(1, DM),
        bd(p['fg2_w']), jnp.tile(p['fg2_b'], H).reshape(1, DM),
        p['proj'], p['n1_g'].reshape(1, DM), p['n1_b'].reshape(1, DM),
        p['fc2_w'], p['fc2_b'].reshape(1, DP),
        p['n2_g'].reshape(1, DP), p['n2_b'].reshape(1, DP),
    )
    attn5, res = _stage_b(q, gkv, rel, features, wp)
    return res, attn5.reshape(B * H, N, K, HD)


# speedup vs baseline: 8.7335x; 1.2512x over previous
import functools
import math

import jax
import jax.numpy as jnp
from jax import lax
from jax.experimental import pallas as pl
from jax.experimental.pallas import tpu as pltpu
from jax.experimental.pallas import tpu_sc as plsc

B, N = 8, 2048
DP, DM, K, H = 64, 128, 16, 4
HD = DM // H

QA = 256
QB = 128
RB = QB * K


def _stage_a_body(xyzq_ref, xyzt_ref, feat_ref, fc1_ref, fc1b_ref,
                  wq_ref, wk_ref, wv_ref, q_ref, tbl_ref, knn_ref, rel_ref):
    b = pl.program_id(0)
    x = jnp.dot(feat_ref[0], fc1_ref[...]) + fc1b_ref[...]
    q_ref[0] = jnp.dot(x, wq_ref[...])
    xk = jnp.dot(x, wk_ref[...])
    xv = jnp.dot(x, wv_ref[...])
    tbl_ref[0] = jnp.concatenate([xk, xv], axis=1)

    xq = xyzq_ref[0]
    xt = xyzt_ref[0]
    qn = jnp.sum(xq * xq, axis=1, keepdims=True)
    kn = jnp.sum(xt * xt, axis=0, keepdims=True)
    cross = lax.dot_general(xq, xt, (((1,), (0,)), ((), ())),
                            precision=lax.Precision.DEFAULT)
    d = qn + kn - 2.0 * cross

    iota = lax.broadcasted_iota(jnp.int32, (QA, N), 1)
    cols = []
    rels = []
    for j in range(K):
        m = jnp.min(d, axis=1, keepdims=True)
        idx = jnp.min(jnp.where(d == m, iota, N), axis=1, keepdims=True)
        cols.append(idx)
        hit = iota == idx
        nbx = lax.dot_general(jnp.where(hit, 1.0, 0.0), xt,
                              (((1,), (1,)), ((), ())),
                              precision=lax.Precision.HIGHEST)
        rels.append(xq - nbx)
        d = jnp.where(hit, jnp.float32(jnp.inf), d)
    knn_ref[0] = jnp.concatenate(cols, axis=1) + b * N
    rel_ref[0] = jnp.concatenate(rels, axis=1)


def _stage_a(xyz, xyzt, feats, p):
    grid = (B, N // QA)
    out = pl.pallas_call(
        _stage_a_body,
        grid=grid,
        in_specs=[
            pl.BlockSpec((1, QA, 3), lambda b, i: (b, i, 0)),
            pl.BlockSpec((1, 3, N), lambda b, i: (b, 0, 0)),
            pl.BlockSpec((1, QA, DP), lambda b, i: (b, i, 0)),
            pl.BlockSpec((DP, DM), lambda b, i: (0, 0)),
            pl.BlockSpec((1, DM), lambda b, i: (0, 0)),
            pl.BlockSpec((DM, DM), lambda b, i: (0, 0)),
            pl.BlockSpec((DM, DM), lambda b, i: (0, 0)),
            pl.BlockSpec((DM, DM), lambda b, i: (0, 0)),
        ],
        out_specs=[
            pl.BlockSpec((1, QA, DM), lambda b, i: (b, i, 0)),
            pl.BlockSpec((1, QA, 2 * DM), lambda b, i: (b, i, 0)),
            pl.BlockSpec((1, QA, K), lambda b, i: (b, i, 0)),
            pl.BlockSpec((1, QA, 3 * K), lambda b, i: (b, i, 0)),
        ],
        out_shape=[
            jax.ShapeDtypeStruct((B, N, DM), jnp.float32),
            jax.ShapeDtypeStruct((B, N, 2 * DM), jnp.float32),
            jax.ShapeDtypeStruct((B, N, K), jnp.int32),
            jax.ShapeDtypeStruct((B, N, 3 * K), jnp.float32),
        ],
    )(xyz, xyzt, feats, p['fc1_w'], p['fc1_b'].reshape(1, DM),
      p['wq'], p['wk'], p['wv'])
    return out


def _sc_gather(tbl2, idx3, nc, nw):
    tot = B * N * K
    per_w = tot // nw
    ch = 128
    nch = per_w // ch

    mesh = plsc.VectorSubcoreMesh(core_axis_name="c", subcore_axis_name="s")

    @functools.partial(
        pl.kernel,
        mesh=mesh,
        out_type=jax.ShapeDtypeStruct((tot, 2 * DM), jnp.float32),
        scratch_types=[
            pltpu.VMEM((nch, ch), jnp.int32),
            pltpu.VMEM((ch, 2 * DM), jnp.float32),
            pltpu.VMEM((ch, 2 * DM), jnp.float32),
            pltpu.SemaphoreType.DMA,
            pltpu.SemaphoreType.DMA,
        ],
    )
    def k(tbl_hbm, idx_hbm, gkv_hbm, idx_v, buf0, buf1, s0, s1):
        wid = lax.axis_index("s") * nc + lax.axis_index("c")
        pltpu.sync_copy(idx_hbm.at[wid], idx_v)
        bufs = (buf0, buf1)
        sems = (s0, s1)
        pltpu.async_copy(tbl_hbm.at[idx_v.at[0]], buf0, s0)

        def body(i, carry):
            for t in range(2):
                c = 2 * i + t
                buf, sem = bufs[t], sems[t]
                nbuf, nsem = bufs[1 - t], sems[1 - t]
                nc_ = c + 1

                @pl.when(nc_ < nch)
                def _():
                    pltpu.async_copy(tbl_hbm.at[idx_v.at[nc_]], nbuf, nsem)

                pltpu.make_async_copy(tbl_hbm.at[idx_v.at[c]], buf, sem).wait()
                row0 = wid * per_w + c * ch
                pltpu.sync_copy(buf, gkv_hbm.at[pl.ds(row0, ch)])
            return carry

        lax.fori_loop(0, nch // 2, body, 0)

    return k(tbl2, idx3)


def _layernorm(x, g, b):
    m = jnp.mean(x, axis=1, keepdims=True)
    c = x - m
    v = jnp.mean(c * c, axis=1, keepdims=True)
    return c * lax.rsqrt(v + 1e-5) * g + b


def _stage_b_body(q_ref, gkv_ref, rel_ref, feat_ref,
                  fd1_ref, fd1b_ref, fd2_ref, fd2b_ref,
                  fg1_ref, fg1b_ref, fg2_ref, fg2b_ref,
                  proj_ref, n1g_ref, n1b_ref,
                  fc2_ref, fc2b_ref, n2g_ref, n2b_ref,
                  attn_ref, res_ref):
    kk = gkv_ref[:, :DM]
    vv = gkv_ref[:, DM:]

    rel = rel_ref[0]
    p1 = jax.nn.relu(jnp.dot(rel, fd1_ref[...]) + fd1b_ref[...])
    p1 = p1.reshape(QB, K, DM).reshape(RB, DM)
    pos = jnp.dot(p1, fd2_ref[...]) + fd2b_ref[...]

    qq = jnp.broadcast_to(q_ref[0][:, None, :], (QB, K, DM)).reshape(RB, DM)
    h = qq - kk + pos
    a = jax.nn.relu(jnp.dot(h, fg1_ref[...]) + fg1b_ref[...])
    a = jnp.dot(a, fg2_ref[...]) + fg2b_ref[...]
    a = a * jnp.float32(1.0 / math.sqrt(HD))
    a3 = a.reshape(QB, K, DM)
    m = jnp.max(a3, axis=1, keepdims=True)
    e = jnp.exp(a3 - m)
    s = jnp.sum(e, axis=1, keepdims=True)
    attn = e / s

    a_t = jnp.transpose(attn.reshape(QB, K * DM), (1, 0)).reshape(K, DM, QB)
    for hh in range(H):
        attn_ref[0, hh] = a_t[:, hh * HD:(hh + 1) * HD, :]

    w = attn * (vv + pos).reshape(QB, K, DM)
    resv = jnp.sum(w, axis=1)
    r1 = _layernorm(jnp.dot(resv, proj_ref[...]), n1g_ref[...], n1b_ref[...])
    r2 = jnp.dot(r1, fc2_ref[...]) + fc2b_ref[...]
    r2 = _layernorm(r2, n2g_ref[...], n2b_ref[...]) + feat_ref[0]
    res_ref[0] = jnp.transpose(r2, (1, 0))


def _stage_b(q, gkv, rel, feats, wp):
    grid = (B, N // QB)
    nb = N // QB
    full = lambda r, c: pl.BlockSpec((r, c), lambda b, i: (0, 0))
    out = pl.pallas_call(
        _stage_b_body,
        grid=grid,
        in_specs=[
            pl.BlockSpec((1, QB, DM), lambda b, i: (b, i, 0)),
            pl.BlockSpec((RB, 2 * DM), lambda b, i: (b * nb + i, 0)),
            pl.BlockSpec((1, QB, 3 * K), lambda b, i: (b, i, 0)),
            pl.BlockSpec((1, QB, DP), lambda b, i: (b, i, 0)),
            full(3 * K, K * DM), full(1, K * DM), full(DM, DM), full(1, DM),
            full(DM, DM), full(1, DM), full(DM, DM), full(1, DM),
            full(DM, DM), full(1, DM), full(1, DM),
            full(DM, DP), full(1, DP), full(1, DP), full(1, DP),
        ],
        out_specs=[
            pl.BlockSpec((1, H, K, HD, QB), lambda b, i: (b, 0, 0, 0, i)),
            pl.BlockSpec((1, DP, QB), lambda b, i: (b, 0, i)),
        ],
        out_shape=[
            jax.ShapeDtypeStruct((B, H, K, HD, N), jnp.float32),
            jax.ShapeDtypeStruct((B, DP, N), jnp.float32),
        ],
    )(q, gkv, rel, feats, *wp)
    return out


def kernel(xyz, features, params):
    p = params
    xyzt = xyz.transpose(0, 2, 1)
    q, tbl, knn, rel = _stage_a(xyz, xyzt, features, p)

    info = plsc.get_sparse_core_info()
    nc, ns = info.num_cores, info.num_subcores
    nw = nc * ns
    tbl2 = tbl.reshape(B * N, 2 * DM)
    idx3 = knn.reshape(nw, (B * N * K) // (nw * 128), 128)
    gkv = _sc_gather(tbl2, idx3, nc, nw)

    z = jnp.zeros((HD, HD), jnp.float32)
    def bd(w):
        rows = []
        for i in range(H):
            rows.append(jnp.concatenate(
                [w if i == j else z for j in range(H)], axis=1))
        return jnp.concatenate(rows, axis=0)
    fd1bd = jnp.zeros((3 * K, K * DM), jnp.float32)
    for kk_ in range(K):
        fd1bd = fd1bd.at[3 * kk_:3 * kk_ + 3,
                         DM * kk_:DM * (kk_ + 1)].set(p['fd1_w'])
    wp = (
        fd1bd, jnp.tile(p['fd1_b'], K).reshape(1, K * DM), p['fd2_w'],
        p['fd2_b'].reshape(1, DM),
        bd(p['fg1_w']), jnp.tile(p['fg1_b'], H).reshape(1, DM),
        bd(p['fg2_w']), jnp.tile(p['fg2_b'], H).reshape(1, DM),
        p['proj'], p['n1_g'].reshape(1, DM), p['n1_b'].reshape(1, DM),
        p['fc2_w'], p['fc2_b'].reshape(1, DP),
        p['n2_g'].reshape(1, DP), p['n2_b'].reshape(1, DP),
    )
    attn5, res_t = _stage_b(q, gkv, rel, features, wp)
    attn = attn5.transpose(0, 1, 4, 2, 3).reshape(B * H, N, K, HD)
    res = res_t.transpose(0, 2, 1)
    return res, attn
